# Initial kernel scaffold; baseline (speedup 1.0000x reference)
import jax
import jax.numpy as jnp
from jax.experimental import pallas as pl


def kernel(target_with_bonus_probs, bonus_token_ids, draft_probs, draft_token_ids, uniform_rand, gumbel_noise):
    raise NotImplementedError("write your pallas kernel here")


# speedup vs baseline: 1.3204x; 1.3204x over previous
import jax
import jax.numpy as jnp
from jax.experimental import pallas as pl

_TINY = 1.1754943508222875e-38


def _rs_kernel(t_ref, d_ref, g_ref, ids_ref, unif_ref, bonus_ref, out_ref):
    k = d_ref.shape[1]
    t = t_ref[:, :k, :]
    d = d_ref[...]
    g = g_ref[...]

    f = jnp.maximum(t - d, _TINY)
    score = jnp.log(f) + g
    rec_id = jnp.argmax(score, axis=-1).astype(jnp.int32)

    tid = ids_ref[0]
    lane = jax.lax.broadcasted_iota(jnp.int32, t.shape, 2)
    m = lane == tid[:, :, None]
    sel_t = jnp.sum(jnp.where(m, t, 0.0), axis=-1)
    sel_d = jnp.sum(jnp.where(m, d, 0.0), axis=-1)

    accepted = unif_ref[0] < jnp.minimum(sel_t / sel_d, 1.0)
    not_acc = jnp.logical_not(accepted)
    kidx = jax.lax.broadcasted_iota(jnp.int32, tid.shape, 1)
    limits = jnp.min(jnp.where(not_acc, kidx, k), axis=1)
    acc_mask = kidx < limits[:, None]
    after = kidx == limits[:, None]
    out_k = jnp.where(acc_mask, tid, -1)
    bonus_col = jnp.where(out_k[:, k - 1 : k] != -1, bonus_ref[0], -1)
    out_k = jnp.where(after, rec_id, out_k)
    out_ref[0, :, :k] = out_k
    out_ref[0, :, k:] = bonus_col


@jax.jit
def kernel(target_with_bonus_probs, bonus_token_ids, draft_probs,
           draft_token_ids, uniform_rand, gumbel_noise):
    B, K, V = draft_probs.shape
    RB = 2
    G = B // RB
    ids3 = draft_token_ids.reshape(G, RB, K)
    unif3 = uniform_rand.reshape(G, RB, K)
    bonus3 = bonus_token_ids.reshape(G, RB, 1)
    out = pl.pallas_call(
        _rs_kernel,
        grid=(G,),
        in_specs=[
            pl.BlockSpec((RB, K + 1, V), lambda i: (i, 0, 0)),
            pl.BlockSpec((RB, K, V), lambda i: (i, 0, 0)),
            pl.BlockSpec((RB, K, V), lambda i: (i, 0, 0)),
            pl.BlockSpec((1, RB, K), lambda i: (i, 0, 0)),
            pl.BlockSpec((1, RB, K), lambda i: (i, 0, 0)),
            pl.BlockSpec((1, RB, 1), lambda i: (i, 0, 0)),
        ],
        out_specs=pl.BlockSpec((1, RB, K + 1), lambda i: (i, 0, 0)),
        out_shape=jax.ShapeDtypeStruct((G, RB, K + 1), jnp.int32),
    )(target_with_bonus_probs, draft_probs, gumbel_noise, ids3, unif3, bonus3)
    return out.reshape(B, K + 1)


# speedup vs baseline: 1.4401x; 1.0907x over previous
import jax
import jax.numpy as jnp
from jax.experimental import pallas as pl
from jax.experimental.pallas import tpu as pltpu

_TINY = 1.1754943508222875e-38


def _rs_kernel(ids_smem, unif_smem, bonus_smem, t_ref, d_ref, g_ref, ids_ref,
               out_ref):
    k = d_ref.shape[1]
    rb = d_ref.shape[0]
    t = t_ref[:, :k, :]
    d = d_ref[...]
    g = g_ref[...]

    f = jnp.maximum(t - d, _TINY)
    score = jnp.log(f) + g
    rec_id = jnp.argmax(score, axis=-1).astype(jnp.int32)

    lane = jax.lax.broadcasted_iota(jnp.int32, (1, 128), 1)
    kidx = jax.lax.broadcasted_iota(jnp.int32, (rb, k), 1)
    ridx = jax.lax.broadcasted_iota(jnp.int32, (rb, k), 0)
    accepted = jnp.zeros((rb, k), jnp.int32)
    for r in range(rb):
        for kk in range(k):
            tid_s = ids_smem[0, r, kk]
            base = pl.multiple_of((tid_s // 128) * 128, 128)
            tv = t_ref[r, kk : kk + 1, pl.ds(base, 128)]
            dv = d_ref[r, kk : kk + 1, pl.ds(base, 128)]
            msk = lane == (tid_s - base)
            sel_t = jnp.sum(jnp.where(msk, tv, 0.0), axis=1, keepdims=True)
            sel_d = jnp.sum(jnp.where(msk, dv, 0.0), axis=1, keepdims=True)
            acc = jnp.where(
                unif_smem[0, r, kk] < jnp.minimum(sel_t / sel_d, 1.0), 1, 0
            ).astype(jnp.int32)
            accepted = jnp.where((ridx == r) & (kidx == kk), acc, accepted)

    not_acc = accepted == 0
    limits = jnp.min(jnp.where(not_acc, kidx, k), axis=1)
    acc_mask = kidx < limits[:, None]
    after = kidx == limits[:, None]
    tid = ids_ref[0]
    out_k = jnp.where(acc_mask, tid, -1)
    rcol = jax.lax.broadcasted_iota(jnp.int32, (rb, 1), 0)
    bonus_vec = jnp.zeros((rb, 1), jnp.int32)
    for r in range(rb):
        bonus_vec = jnp.where(rcol == r, bonus_smem[0, r, 0], bonus_vec)
    bonus_col = jnp.where(out_k[:, k - 1 : k] != -1, bonus_vec, -1)
    out_k = jnp.where(after, rec_id, out_k)
    out_ref[0, :, :k] = out_k
    out_ref[0, :, k:] = bonus_col


@jax.jit
def kernel(target_with_bonus_probs, bonus_token_ids, draft_probs,
           draft_token_ids, uniform_rand, gumbel_noise):
    B, K, V = draft_probs.shape
    RB = 2
    G = B // RB
    ids3 = draft_token_ids.reshape(G, RB, K)
    unif3 = uniform_rand.reshape(G, RB, K)
    bonus3 = bonus_token_ids.reshape(G, RB, 1)
    out = pl.pallas_call(
        _rs_kernel,
        grid=(G,),
        in_specs=[
            pl.BlockSpec((1, RB, K), lambda i: (i, 0, 0),
                         memory_space=pltpu.SMEM),
            pl.BlockSpec((1, RB, K), lambda i: (i, 0, 0),
                         memory_space=pltpu.SMEM),
            pl.BlockSpec((1, RB, 1), lambda i: (i, 0, 0),
                         memory_space=pltpu.SMEM),
            pl.BlockSpec((RB, K + 1, V), lambda i: (i, 0, 0)),
            pl.BlockSpec((RB, K, V), lambda i: (i, 0, 0)),
            pl.BlockSpec((RB, K, V), lambda i: (i, 0, 0)),
            pl.BlockSpec((1, RB, K), lambda i: (i, 0, 0)),
        ],
        out_specs=pl.BlockSpec((1, RB, K + 1), lambda i: (i, 0, 0)),
        out_shape=jax.ShapeDtypeStruct((G, RB, K + 1), jnp.int32),
    )(ids3, unif3, bonus3, target_with_bonus_probs, draft_probs, gumbel_noise,
      ids3)
    return out.reshape(B, K + 1)
